# Initial kernel scaffold; baseline (speedup 1.0000x reference)
import jax
import jax.numpy as jnp
from jax.experimental import pallas as pl


def kernel(x, conv0_w, conv0_b, conv1_w, conv1_b, conv2_w, conv2_b, conv3_w, conv3_b, conv4_w, conv4_b, fc1_w, fc1_b, fc2_w, fc2_b):
    raise NotImplementedError("write your pallas kernel here")


# speedup vs baseline: 1.0557x; 1.0557x over previous
import functools

import jax
import jax.numpy as jnp
from jax.experimental import pallas as pl
from jax.experimental.pallas import tpu as pltpu


def _conv_kernel(z_ref, w_ref, b_ref, o_ref, *, wo_out, o_ch):
    f32 = jnp.float32
    z = z_ref[0]
    M = z.shape[1]
    O = o_ch

    t = jnp.dot(w_ref[...], z, preferred_element_type=f32)
    ya = t[0:O]
    yb = t[O:2 * O]
    yc = t[2 * O:3 * O]
    yd = t[3 * O:4 * O]

    col = jax.lax.broadcasted_iota(jnp.int32, (1, M), 1)
    wo_is0 = (col & (wo_out - 1)) == 0

    sb = jnp.where(
        wo_is0, 0.0,
        jnp.concatenate([jnp.zeros((O, 1), f32), yb[:, :M - 1]], axis=1))
    sc = jnp.concatenate(
        [jnp.zeros((O, wo_out), f32), yc[:, :M - wo_out]], axis=1)
    sd = jnp.where(
        wo_is0, 0.0,
        jnp.concatenate([jnp.zeros((O, wo_out + 1), f32),
                         yd[:, :M - wo_out - 1]], axis=1))

    acc = ya + sb + sc + sd + b_ref[...]
    o_ref[0] = jnp.maximum(acc, 0.0).astype(o_ref.dtype)


def _conv_layer(z, w_packed, b_col, *, wo_out, out_dtype):
    B, c4, M = z.shape
    O = w_packed.shape[0] // 4
    body = functools.partial(_conv_kernel, wo_out=wo_out, o_ch=O)
    return pl.pallas_call(
        body,
        out_shape=jax.ShapeDtypeStruct((B, O, M), out_dtype),
        grid_spec=pltpu.PrefetchScalarGridSpec(
            num_scalar_prefetch=0,
            grid=(B,),
            in_specs=[
                pl.BlockSpec((1, c4, M), lambda b: (b, 0, 0)),
                pl.BlockSpec((4 * O, c4), lambda b: (0, 0)),
                pl.BlockSpec((O, 1), lambda b: (0, 0)),
            ],
            out_specs=pl.BlockSpec((1, O, M), lambda b: (b, 0, 0)),
        ),
        compiler_params=pltpu.CompilerParams(
            dimension_semantics=("parallel",),
            vmem_limit_bytes=64 * 1024 * 1024),
    )(z, w_packed, b_col)


def _fc_kernel(x_ref, w1_ref, b1_ref, w2_ref, b2_ref, o_ref):
    h = jnp.dot(x_ref[...], w1_ref[...],
                preferred_element_type=jnp.float32) + b1_ref[...]
    h = jnp.where(h > 0.0, h, 0.2 * h)
    o_ref[...] = jnp.dot(h.astype(jnp.bfloat16), w2_ref[...],
                         preferred_element_type=jnp.float32) + b2_ref[...]


def _fc_head(feat, w1, b1, w2, b2):
    B = feat.shape[0]
    out_dim = w2.shape[0]
    return pl.pallas_call(
        _fc_kernel,
        out_shape=jax.ShapeDtypeStruct((B, out_dim), jnp.float32),
    )(feat, w1.T.astype(jnp.bfloat16), b1.reshape(1, -1),
      w2.T.astype(jnp.bfloat16), b2.reshape(1, -1))


def _space_to_depth(x):
    B, C, H, W = x.shape
    t = x.reshape(B, C, H // 2, 2, W // 2, 2)
    t = jnp.transpose(t, (0, 3, 5, 1, 2, 4))
    return t.reshape(B, 4 * C, (H // 2) * (W // 2))


def _pack_conv_weight(w):
    O, C = w.shape[0], w.shape[1]
    z = jnp.zeros((O, C), w.dtype)
    t = lambda kh, kw: w[:, :, kh, kw]
    wa = jnp.concatenate([t(1, 1), t(1, 2), t(2, 1), t(2, 2)], axis=1)
    wb = jnp.concatenate([z,       t(1, 0), z,       t(2, 0)], axis=1)
    wc = jnp.concatenate([z,       z,       t(0, 1), t(0, 2)], axis=1)
    wd = jnp.concatenate([z,       z,       z,       t(0, 0)], axis=1)
    return jnp.concatenate([wa, wb, wc, wd], axis=0)


def kernel(x, conv0_w, conv0_b, conv1_w, conv1_b, conv2_w, conv2_b,
           conv3_w, conv3_b, conv4_w, conv4_b, fc1_w, fc1_b, fc2_w, fc2_b):
    bf16 = jnp.bfloat16
    B = x.shape[0]
    conv_ws = [conv0_w, conv1_w, conv2_w, conv3_w, conv4_w]
    conv_bs = [conv0_b, conv1_b, conv2_b, conv3_b, conv4_b]

    z = _space_to_depth(x.astype(bf16))
    spatial = x.shape[2] // 2
    out = None
    for i in range(5):
        w = conv_ws[i]
        b = conv_bs[i]
        O = w.shape[0]
        out_dtype = bf16 if i < 4 else bf16
        out = _conv_layer(z, _pack_conv_weight(w).astype(bf16),
                          b.reshape(O, 1).astype(jnp.float32),
                          wo_out=spatial, out_dtype=out_dtype)
        if i < 4:
            z = _space_to_depth(out.reshape(B, O, spatial, spatial))
            spatial //= 2
    feat = out.reshape(B, 2048)
    return _fc_head(feat, fc1_w, fc1_b, fc2_w, fc2_b)


# speedup vs baseline: 2.1847x; 2.0694x over previous
import jax
import jax.numpy as jnp
from jax.experimental import pallas as pl
from jax.experimental.pallas import tpu as pltpu


def _space_to_depth_nchw(x):
    B, C, H, W = x.shape
    t = x.reshape(B, C, H // 2, 2, W // 2, 2)
    t = jnp.transpose(t, (0, 3, 5, 1, 2, 4))
    return t.reshape(B, 4 * C, (H // 2) * (W // 2))


def _pack_conv_weight(w):
    O, C = w.shape[0], w.shape[1]
    z = jnp.zeros((O, C), w.dtype)
    t = lambda kh, kw: w[:, :, kh, kw]
    wa = jnp.concatenate([t(1, 1), t(1, 2), t(2, 1), t(2, 2)], axis=1)
    wb = jnp.concatenate([z,       t(1, 0), z,       t(2, 0)], axis=1)
    wc = jnp.concatenate([z,       z,       t(0, 1), t(0, 2)], axis=1)
    wd = jnp.concatenate([z,       z,       z,       t(0, 0)], axis=1)
    return jnp.concatenate([wa, wb, wc, wd], axis=0)


def _perm_cols(src):
    n = src.shape[0]
    return jnp.eye(n, dtype=jnp.float32)[:, src]


def _packed_perm(S, k):
    n = k * S
    t = jnp.arange(n)
    blk = t // (n // 4)
    p, q = blk // 2, blk % 2
    r = t % (n // 4)
    j, w_ = r // (S // 2), r % (S // 2)
    return _perm_cols((2 * j + p) * S + 2 * w_ + q)


def _flat_perm(S):
    n = S * S
    t = jnp.arange(n)
    blk = t // (n // 4)
    p, q = blk // 2, blk % 2
    r = t % (n // 4)
    h_, w_ = r // (S // 2), r % (S // 2)
    return _perm_cols((2 * h_ + p) * S + 2 * w_ + q)


def _conv_block(z, w_packed, b_col, wo_out):
    f32 = jnp.float32
    M = z.shape[1]
    O = w_packed.shape[0] // 4
    t = jnp.dot(w_packed, z, preferred_element_type=f32)
    ya = t[0:O]
    yb = t[O:2 * O]
    yc = t[2 * O:3 * O]
    yd = t[3 * O:4 * O]
    col = jax.lax.broadcasted_iota(jnp.int32, (1, M), 1)
    wo_is0 = (col & (wo_out - 1)) == 0
    sb = jnp.where(
        wo_is0, 0.0,
        jnp.concatenate([jnp.zeros((O, 1), f32), yb[:, :M - 1]], axis=1))
    sc = jnp.concatenate(
        [jnp.zeros((O, wo_out), f32), yc[:, :M - wo_out]], axis=1)
    sd = jnp.where(
        wo_is0, 0.0,
        jnp.concatenate([jnp.zeros((O, wo_out + 1), f32),
                         yd[:, :M - wo_out - 1]], axis=1))
    acc = ya + sb + sc + sd + b_col
    return jnp.maximum(acc, 0.0)


def _fused_kernel(z_ref, w0, b0, w1, b1, w2, b2, w3, b3, w4, b4,
                  d0, d1, d2, d3, w1big, diagm, b1r, w2t, b2r, o_ref):
    f32 = jnp.float32
    dot = lambda a, b: jnp.dot(a, b, preferred_element_type=f32)

    z = z_ref[0]
    y = _conv_block(z, w0[...], b0[...], 128)

    a = dot(y.reshape(512, 512), d0[...])
    z = jnp.concatenate(
        [a[:, k * 128:(k + 1) * 128].reshape(16, 4096) for k in range(4)], 0)
    y = _conv_block(z, w1[...], b1[...], 64)

    a = dot(y.reshape(256, 512), d1[...])
    z = jnp.concatenate(
        [a[:, k * 128:(k + 1) * 128].reshape(32, 1024) for k in range(4)], 0)
    y = _conv_block(z, w2[...], b2[...], 32)

    a = dot(y, d2[...])
    z = jnp.concatenate([a[:, k * 256:(k + 1) * 256] for k in range(4)], 0)
    y = _conv_block(z, w3[...], b3[...], 16)

    a = dot(y, d3[...])
    z = jnp.concatenate([a[:, k * 64:(k + 1) * 64] for k in range(4)], 0)
    y = _conv_block(z, w4[...], b4[...], 8)

    v = dot(y, w1big[...]) * diagm[...]
    u = jnp.sum(v, axis=0, keepdims=True).reshape(16, 128)
    u = jnp.sum(u, axis=0, keepdims=True)
    h = u[:, :64] + u[:, 64:] + b1r[...]
    h = jnp.where(h > 0.0, h, 0.2 * h)
    o_ref[0] = dot(h, w2t[...]) + b2r[...]


def kernel(x, conv0_w, conv0_b, conv1_w, conv1_b, conv2_w, conv2_b,
           conv3_w, conv3_b, conv4_w, conv4_b, fc1_w, fc1_b, fc2_w, fc2_b):
    f32 = jnp.float32
    B = x.shape[0]
    out_dim = fc2_w.shape[0]
    z0 = _space_to_depth_nchw(x.astype(f32))

    conv_ws = [conv0_w, conv1_w, conv2_w, conv3_w, conv4_w]
    conv_bs = [conv0_b, conv1_b, conv2_b, conv3_b, conv4_b]
    packed = [_pack_conv_weight(w).astype(f32) for w in conv_ws]
    bcols = [b.reshape(-1, 1).astype(f32) for b in conv_bs]

    d_mats = [_packed_perm(128, 4), _packed_perm(64, 8),
              _flat_perm(32), _flat_perm(16)]

    w1big = jnp.transpose(fc1_w.reshape(64, 32, 64), (2, 1, 0)).reshape(64, 2048)
    diagm = jnp.repeat(jnp.eye(32, dtype=f32), 64, axis=1)

    full = lambda arr: pl.BlockSpec(arr.shape, lambda b: (0,) * arr.ndim)
    operands = [z0]
    in_specs = [pl.BlockSpec((1, 12, 16384), lambda b: (b, 0, 0))]
    for wp, bc in zip(packed, bcols):
        operands += [wp, bc]
        in_specs += [full(wp), full(bc)]
    for d in d_mats:
        operands.append(d)
        in_specs.append(full(d))
    tail = [w1big, diagm, fc1_b.reshape(1, 64).astype(f32),
            fc2_w.T.astype(f32), fc2_b.reshape(1, out_dim).astype(f32)]
    for t in tail:
        operands.append(t)
        in_specs.append(full(t))

    out = pl.pallas_call(
        _fused_kernel,
        out_shape=jax.ShapeDtypeStruct((B, 1, out_dim), f32),
        grid_spec=pltpu.PrefetchScalarGridSpec(
            num_scalar_prefetch=0,
            grid=(B,),
            in_specs=in_specs,
            out_specs=pl.BlockSpec((1, 1, out_dim), lambda b: (b, 0, 0)),
        ),
        compiler_params=pltpu.CompilerParams(
            dimension_semantics=("parallel",),
            vmem_limit_bytes=100 * 1024 * 1024),
    )(*operands)
    return out.reshape(B, out_dim)


# speedup vs baseline: 2.6988x; 1.2353x over previous
import jax
import jax.numpy as jnp
from jax.experimental import pallas as pl
from jax.experimental.pallas import tpu as pltpu


def _space_to_depth_nchw(x):
    B, C, H, W = x.shape
    t = x.reshape(B, C, H // 2, 2, W // 2, 2)
    t = jnp.transpose(t, (0, 3, 5, 1, 2, 4))
    return t.reshape(B, 4 * C, (H // 2) * (W // 2))


def _pack_conv_weight(w):
    O, C = w.shape[0], w.shape[1]
    z = jnp.zeros((O, C), w.dtype)
    t = lambda kh, kw: w[:, :, kh, kw]
    wa = jnp.concatenate([t(1, 1), t(1, 2), t(2, 1), t(2, 2)], axis=1)
    wb = jnp.concatenate([z,       t(1, 0), z,       t(2, 0)], axis=1)
    wc = jnp.concatenate([z,       z,       t(0, 1), t(0, 2)], axis=1)
    wd = jnp.concatenate([z,       z,       z,       t(0, 0)], axis=1)
    return jnp.concatenate([wa, wb, wc, wd], axis=0)


def _perm_cols(src):
    n = src.shape[0]
    return jnp.eye(n, dtype=jnp.float32)[:, src]


def _packed_perm(S, k):
    n = k * S
    t = jnp.arange(n)
    blk = t // (n // 4)
    p, q = blk // 2, blk % 2
    r = t % (n // 4)
    j, w_ = r // (S // 2), r % (S // 2)
    return _perm_cols((2 * j + p) * S + 2 * w_ + q)


def _flat_perm(S):
    n = S * S
    t = jnp.arange(n)
    blk = t // (n // 4)
    p, q = blk // 2, blk % 2
    r = t % (n // 4)
    h_, w_ = r // (S // 2), r % (S // 2)
    return _perm_cols((2 * h_ + p) * S + 2 * w_ + q)


def _conv_block(z, w_packed, b_col, wo_out):
    f32 = jnp.float32
    M = z.shape[1]
    O = w_packed.shape[0] // 4
    t = jnp.dot(w_packed, z, preferred_element_type=f32)
    ya = t[0:O]
    yb = t[O:2 * O]
    yc = t[2 * O:3 * O]
    yd = t[3 * O:4 * O]
    col = jax.lax.broadcasted_iota(jnp.int32, (1, M), 1)
    wo_is0 = (col & (wo_out - 1)) == 0
    sb = jnp.where(
        wo_is0, 0.0,
        jnp.concatenate([jnp.zeros((O, 1), f32), yb[:, :M - 1]], axis=1))
    sc = jnp.concatenate(
        [jnp.zeros((O, wo_out), f32), yc[:, :M - wo_out]], axis=1)
    sd = jnp.where(
        wo_is0, 0.0,
        jnp.concatenate([jnp.zeros((O, wo_out + 1), f32),
                         yd[:, :M - wo_out - 1]], axis=1))
    acc = ya + sb + sc + sd + b_col
    return jnp.maximum(acc, 0.0)


def _fused_kernel(x_ref, w0, b0, w1, b1, w2, b2, w3, b3, w4, b4,
                  dx, d0, d1, d2, d3, w1big, diagm, b1r, w2t, b2r, o_ref):
    f32 = jnp.float32
    dot = lambda a, b: jnp.dot(a, b, preferred_element_type=f32)

    xa = x_ref[0].reshape(3, 65536).reshape(384, 512)
    a = dot(xa, dx[...])
    z = jnp.concatenate(
        [a[:, k * 128:(k + 1) * 128].reshape(3, 16384) for k in range(4)], 0)
    y = _conv_block(z, w0[...], b0[...], 128)

    a = dot(y.reshape(512, 512), d0[...])
    z = jnp.concatenate(
        [a[:, k * 128:(k + 1) * 128].reshape(16, 4096) for k in range(4)], 0)
    y = _conv_block(z, w1[...], b1[...], 64)

    a = dot(y.reshape(256, 512), d1[...])
    z = jnp.concatenate(
        [a[:, k * 128:(k + 1) * 128].reshape(32, 1024) for k in range(4)], 0)
    y = _conv_block(z, w2[...], b2[...], 32)

    a = dot(y, d2[...])
    z = jnp.concatenate([a[:, k * 256:(k + 1) * 256] for k in range(4)], 0)
    y = _conv_block(z, w3[...], b3[...], 16)

    a = dot(y, d3[...])
    z = jnp.concatenate([a[:, k * 64:(k + 1) * 64] for k in range(4)], 0)
    y = _conv_block(z, w4[...], b4[...], 8)

    v = dot(y, w1big[...]) * diagm[...]
    u = jnp.sum(v, axis=0, keepdims=True).reshape(16, 128)
    u = jnp.sum(u, axis=0, keepdims=True)
    h = u[:, :64] + u[:, 64:] + b1r[...]
    h = jnp.where(h > 0.0, h, 0.2 * h)
    o_ref[0] = dot(h, w2t[...]) + b2r[...]


def kernel(x, conv0_w, conv0_b, conv1_w, conv1_b, conv2_w, conv2_b,
           conv3_w, conv3_b, conv4_w, conv4_b, fc1_w, fc1_b, fc2_w, fc2_b):
    f32 = jnp.float32
    B = x.shape[0]
    out_dim = fc2_w.shape[0]
    xf = x.astype(f32)

    conv_ws = [conv0_w, conv1_w, conv2_w, conv3_w, conv4_w]
    conv_bs = [conv0_b, conv1_b, conv2_b, conv3_b, conv4_b]
    packed = [_pack_conv_weight(w).astype(f32) for w in conv_ws]
    bcols = [b.reshape(-1, 1).astype(f32) for b in conv_bs]

    d_mats = [_packed_perm(256, 2), _packed_perm(128, 4), _packed_perm(64, 8),
              _flat_perm(32), _flat_perm(16)]

    w1big = jnp.transpose(fc1_w.reshape(64, 32, 64), (2, 1, 0)).reshape(64, 2048)
    diagm = jnp.repeat(jnp.eye(32, dtype=f32), 64, axis=1)

    full = lambda arr: pl.BlockSpec(arr.shape, lambda b: (0,) * arr.ndim)
    operands = [xf]
    in_specs = [pl.BlockSpec((1, 3, 256, 256), lambda b: (b, 0, 0, 0))]
    for wp, bc in zip(packed, bcols):
        operands += [wp, bc]
        in_specs += [full(wp), full(bc)]
    for d in d_mats:
        operands.append(d)
        in_specs.append(full(d))
    tail = [w1big, diagm, fc1_b.reshape(1, 64).astype(f32),
            fc2_w.T.astype(f32), fc2_b.reshape(1, out_dim).astype(f32)]
    for t in tail:
        operands.append(t)
        in_specs.append(full(t))

    out = pl.pallas_call(
        _fused_kernel,
        out_shape=jax.ShapeDtypeStruct((B, 1, out_dim), f32),
        grid_spec=pltpu.PrefetchScalarGridSpec(
            num_scalar_prefetch=0,
            grid=(B,),
            in_specs=in_specs,
            out_specs=pl.BlockSpec((1, 1, out_dim), lambda b: (b, 0, 0)),
        ),
        compiler_params=pltpu.CompilerParams(
            dimension_semantics=("parallel",),
            vmem_limit_bytes=100 * 1024 * 1024),
    )(*operands)
    return out.reshape(B, out_dim)


# speedup vs baseline: 3.3052x; 1.2247x over previous
import jax
import jax.numpy as jnp
from jax.experimental import pallas as pl
from jax.experimental.pallas import tpu as pltpu


def _space_to_depth_nchw(x):
    B, C, H, W = x.shape
    t = x.reshape(B, C, H // 2, 2, W // 2, 2)
    t = jnp.transpose(t, (0, 3, 5, 1, 2, 4))
    return t.reshape(B, 4 * C, (H // 2) * (W // 2))


def _pack_conv_weight(w):
    O, C = w.shape[0], w.shape[1]
    z = jnp.zeros((O, C), w.dtype)
    t = lambda kh, kw: w[:, :, kh, kw]
    wa = jnp.concatenate([t(1, 1), t(1, 2), t(2, 1), t(2, 2)], axis=1)
    wb = jnp.concatenate([z,       t(1, 0), z,       t(2, 0)], axis=1)
    wc = jnp.concatenate([z,       z,       t(0, 1), t(0, 2)], axis=1)
    wd = jnp.concatenate([z,       z,       z,       t(0, 0)], axis=1)
    return jnp.concatenate([wa, wb, wc, wd], axis=0)


def _perm_cols(src):
    n = src.shape[0]
    return jnp.eye(n, dtype=jnp.float32)[:, src]


def _packed_perm(S, k):
    n = k * S
    t = jnp.arange(n)
    blk = t // (n // 4)
    p, q = blk // 2, blk % 2
    r = t % (n // 4)
    j, w_ = r // (S // 2), r % (S // 2)
    return _perm_cols((2 * j + p) * S + 2 * w_ + q)


def _flat_perm(S):
    n = S * S
    t = jnp.arange(n)
    blk = t // (n // 4)
    p, q = blk // 2, blk % 2
    r = t % (n // 4)
    h_, w_ = r // (S // 2), r % (S // 2)
    return _perm_cols((2 * h_ + p) * S + 2 * w_ + q)


def _conv_block(z, w_packed, b_col, wo_out):
    f32 = jnp.float32
    M = z.shape[1]
    O = w_packed.shape[0] // 4
    t = jnp.dot(w_packed, z, preferred_element_type=f32)
    ya = t[0:O]
    yb = t[O:2 * O]
    yc = t[2 * O:3 * O]
    yd = t[3 * O:4 * O]
    col = jax.lax.broadcasted_iota(jnp.int32, (1, M), 1)
    wo_is0 = (col & (wo_out - 1)) == 0
    bd = yb + jnp.concatenate(
        [jnp.zeros((O, wo_out), f32), yd[:, :M - wo_out]], axis=1)
    sbd = jnp.where(
        wo_is0, 0.0,
        jnp.concatenate([jnp.zeros((O, 1), f32), bd[:, :M - 1]], axis=1))
    sc = jnp.concatenate(
        [jnp.zeros((O, wo_out), f32), yc[:, :M - wo_out]], axis=1)
    acc = ya + sbd + sc + b_col
    return jnp.maximum(acc, 0.0).astype(jnp.bfloat16)


_IMGS_PER_PROG = 2


def _fused_kernel(x_ref, w0, b0, w1, b1, w2, b2, w3, b3, w4, b4,
                  dx, d0, d1, d2, d3, w1big, diagm, b1r, w2t, b2r, o_ref):
    f32 = jnp.float32
    dot = lambda a, b: jnp.dot(a, b, preferred_element_type=f32)
    dotb = lambda a, b: jnp.dot(
        a, b, preferred_element_type=f32).astype(jnp.bfloat16)

    gs = range(_IMGS_PER_PROG)
    xa = [x_ref[g].reshape(3, 65536).reshape(384, 512) for g in gs]
    a = [dotb(xa[g], dx[...]) for g in gs]
    z = [jnp.concatenate(
        [a[g][:, k * 128:(k + 1) * 128].reshape(3, 16384) for k in range(4)],
        0) for g in gs]
    y = [_conv_block(z[g], w0[...], b0[...], 128) for g in gs]

    a = [dotb(y[g].reshape(512, 512), d0[...]) for g in gs]
    z = [jnp.concatenate(
        [a[g][:, k * 128:(k + 1) * 128].reshape(16, 4096) for k in range(4)],
        0) for g in gs]
    y = [_conv_block(z[g], w1[...], b1[...], 64) for g in gs]

    a = [dotb(y[g].reshape(256, 512), d1[...]) for g in gs]
    z = [jnp.concatenate(
        [a[g][:, k * 128:(k + 1) * 128].reshape(32, 1024) for k in range(4)],
        0) for g in gs]
    y = [_conv_block(z[g], w2[...], b2[...], 32) for g in gs]

    a = [dotb(y[g], d2[...]) for g in gs]
    z = [jnp.concatenate(
        [a[g][:, k * 256:(k + 1) * 256] for k in range(4)], 0) for g in gs]
    y = [_conv_block(z[g], w3[...], b3[...], 16) for g in gs]

    a = [dotb(y[g], d3[...]) for g in gs]
    z = [jnp.concatenate(
        [a[g][:, k * 64:(k + 1) * 64] for k in range(4)], 0) for g in gs]
    y = [_conv_block(z[g], w4[...], b4[...], 8) for g in gs]

    for g in gs:
        v = dot(y[g], w1big[...]) * diagm[...]
        u = jnp.sum(v, axis=0, keepdims=True).reshape(16, 128)
        u = jnp.sum(u, axis=0, keepdims=True)
        h = u[:, :64] + u[:, 64:] + b1r[...]
        h = jnp.where(h > 0.0, h, 0.2 * h)
        o_ref[g] = dot(h.astype(jnp.bfloat16), w2t[...]) + b2r[...]


def kernel(x, conv0_w, conv0_b, conv1_w, conv1_b, conv2_w, conv2_b,
           conv3_w, conv3_b, conv4_w, conv4_b, fc1_w, fc1_b, fc2_w, fc2_b):
    f32 = jnp.float32
    bf16 = jnp.bfloat16
    B = x.shape[0]
    out_dim = fc2_w.shape[0]
    xf = x.astype(bf16)

    conv_ws = [conv0_w, conv1_w, conv2_w, conv3_w, conv4_w]
    conv_bs = [conv0_b, conv1_b, conv2_b, conv3_b, conv4_b]
    packed = [_pack_conv_weight(w).astype(bf16) for w in conv_ws]
    bcols = [b.reshape(-1, 1).astype(f32) for b in conv_bs]

    d_mats = [m.astype(bf16) for m in
              [_packed_perm(256, 2), _packed_perm(128, 4), _packed_perm(64, 8),
               _flat_perm(32), _flat_perm(16)]]

    w1big = jnp.transpose(fc1_w.reshape(64, 32, 64), (2, 1, 0)).reshape(64, 2048)
    diagm = jnp.repeat(jnp.eye(32, dtype=f32), 64, axis=1)

    G = _IMGS_PER_PROG
    full = lambda arr: pl.BlockSpec(arr.shape, lambda b: (0,) * arr.ndim)
    operands = [xf]
    in_specs = [pl.BlockSpec((G, 3, 256, 256), lambda b: (b, 0, 0, 0))]
    for wp, bc in zip(packed, bcols):
        operands += [wp, bc]
        in_specs += [full(wp), full(bc)]
    for d in d_mats:
        operands.append(d)
        in_specs.append(full(d))
    tail = [w1big.astype(bf16), diagm, fc1_b.reshape(1, 64).astype(f32),
            fc2_w.T.astype(bf16), fc2_b.reshape(1, out_dim).astype(f32)]
    for t in tail:
        operands.append(t)
        in_specs.append(full(t))

    out = pl.pallas_call(
        _fused_kernel,
        out_shape=jax.ShapeDtypeStruct((B, 1, out_dim), f32),
        grid_spec=pltpu.PrefetchScalarGridSpec(
            num_scalar_prefetch=0,
            grid=(B // G,),
            in_specs=in_specs,
            out_specs=pl.BlockSpec((G, 1, out_dim), lambda b: (b, 0, 0)),
        ),
        compiler_params=pltpu.CompilerParams(
            dimension_semantics=("parallel",),
            vmem_limit_bytes=100 * 1024 * 1024),
    )(*operands)
    return out.reshape(B, out_dim)


# speedup vs baseline: 4.0961x; 1.2393x over previous
import jax
import jax.numpy as jnp
from jax.experimental import pallas as pl
from jax.experimental.pallas import tpu as pltpu


def _space_to_depth_nchw(x):
    B, C, H, W = x.shape
    t = x.reshape(B, C, H // 2, 2, W // 2, 2)
    t = jnp.transpose(t, (0, 3, 5, 1, 2, 4))
    return t.reshape(B, 4 * C, (H // 2) * (W // 2))


def _pack_conv_weight(w):
    O, C = w.shape[0], w.shape[1]
    z = jnp.zeros((O, C), w.dtype)
    t = lambda kh, kw: w[:, :, kh, kw]
    wa = jnp.concatenate([t(1, 1), t(1, 2), t(2, 1), t(2, 2)], axis=1)
    wb = jnp.concatenate([z,       t(1, 0), z,       t(2, 0)], axis=1)
    wc = jnp.concatenate([z,       z,       t(0, 1), t(0, 2)], axis=1)
    wd = jnp.concatenate([z,       z,       z,       t(0, 0)], axis=1)
    return jnp.concatenate([wa, wb, wc, wd], axis=0)


def _perm_cols(src):
    n = src.shape[0]
    return jnp.eye(n, dtype=jnp.float32)[:, src]


def _packed_perm(S, k):
    n = k * S
    t = jnp.arange(n)
    blk = t // (n // 4)
    p, q = blk // 2, blk % 2
    r = t % (n // 4)
    j, w_ = r // (S // 2), r % (S // 2)
    return _perm_cols((2 * j + p) * S + 2 * w_ + q)


def _flat_perm(S):
    n = S * S
    t = jnp.arange(n)
    blk = t // (n // 4)
    p, q = blk // 2, blk % 2
    r = t % (n // 4)
    h_, w_ = r // (S // 2), r % (S // 2)
    return _perm_cols((2 * h_ + p) * S + 2 * w_ + q)


def _conv_block(z, w_packed, b_col, wo_out):
    f32 = jnp.float32
    M = z.shape[1]
    O = w_packed.shape[0] // 4
    t = jnp.dot(w_packed, z, preferred_element_type=f32)
    ya = t[0:O]
    yb = t[O:2 * O]
    yc = t[2 * O:3 * O]
    yd = t[3 * O:4 * O]
    col = jax.lax.broadcasted_iota(jnp.int32, (1, M), 1)
    wo_is0 = (col & (wo_out - 1)) == 0
    bd = yb + jnp.concatenate(
        [jnp.zeros((O, wo_out), f32), yd[:, :M - wo_out]], axis=1)
    sbd = jnp.where(
        wo_is0, 0.0,
        jnp.concatenate([jnp.zeros((O, 1), f32), bd[:, :M - 1]], axis=1))
    sc = jnp.concatenate(
        [jnp.zeros((O, wo_out), f32), yc[:, :M - wo_out]], axis=1)
    acc = ya + sbd + sc + b_col
    return jnp.maximum(acc, 0.0).astype(jnp.bfloat16)


_IMGS_PER_PROG = 4


def _fused_kernel(x_ref, w0, b0, w1, b1, w2, b2, w3, b3, w4, b4,
                  dx, d0, d1, d2, d3, w1big, diagm, b1r, w2t, b2r, o_ref):
    f32 = jnp.float32
    dot = lambda a, b: jnp.dot(a, b, preferred_element_type=f32)
    dotb = lambda a, b: jnp.dot(
        a, b, preferred_element_type=f32).astype(jnp.bfloat16)

    gs = range(_IMGS_PER_PROG)
    xa = [x_ref[g].astype(jnp.bfloat16).reshape(3, 65536).reshape(384, 512)
          for g in gs]
    a = [dotb(xa[g], dx[...]) for g in gs]
    z = [jnp.concatenate(
        [a[g][:, k * 128:(k + 1) * 128].reshape(3, 16384) for k in range(4)],
        0) for g in gs]
    y = [_conv_block(z[g], w0[...], b0[...], 128) for g in gs]

    a = [dotb(y[g].reshape(512, 512), d0[...]) for g in gs]
    z = [jnp.concatenate(
        [a[g][:, k * 128:(k + 1) * 128].reshape(16, 4096) for k in range(4)],
        0) for g in gs]
    y = [_conv_block(z[g], w1[...], b1[...], 64) for g in gs]

    a = [dotb(y[g].reshape(256, 512), d1[...]) for g in gs]
    z = [jnp.concatenate(
        [a[g][:, k * 128:(k + 1) * 128].reshape(32, 1024) for k in range(4)],
        0) for g in gs]
    y = [_conv_block(z[g], w2[...], b2[...], 32) for g in gs]

    a = [dotb(y[g], d2[...]) for g in gs]
    z = [jnp.concatenate(
        [a[g][:, k * 256:(k + 1) * 256] for k in range(4)], 0) for g in gs]
    y = [_conv_block(z[g], w3[...], b3[...], 16) for g in gs]

    a = [dotb(y[g], d3[...]) for g in gs]
    z = [jnp.concatenate(
        [a[g][:, k * 64:(k + 1) * 64] for k in range(4)], 0) for g in gs]
    y = [_conv_block(z[g], w4[...], b4[...], 8) for g in gs]

    for g in gs:
        v = dot(y[g], w1big[...]) * diagm[...]
        u = jnp.sum(v, axis=0, keepdims=True).reshape(16, 128)
        u = jnp.sum(u, axis=0, keepdims=True)
        h = u[:, :64] + u[:, 64:] + b1r[...]
        h = jnp.where(h > 0.0, h, 0.2 * h)
        o_ref[g] = dot(h.astype(jnp.bfloat16), w2t[...]) + b2r[...]


def kernel(x, conv0_w, conv0_b, conv1_w, conv1_b, conv2_w, conv2_b,
           conv3_w, conv3_b, conv4_w, conv4_b, fc1_w, fc1_b, fc2_w, fc2_b):
    f32 = jnp.float32
    bf16 = jnp.bfloat16
    B = x.shape[0]
    out_dim = fc2_w.shape[0]
    xf = x

    conv_ws = [conv0_w, conv1_w, conv2_w, conv3_w, conv4_w]
    conv_bs = [conv0_b, conv1_b, conv2_b, conv3_b, conv4_b]
    packed = [_pack_conv_weight(w).astype(bf16) for w in conv_ws]
    bcols = [b.reshape(-1, 1).astype(f32) for b in conv_bs]

    d_mats = [m.astype(bf16) for m in
              [_packed_perm(256, 2), _packed_perm(128, 4), _packed_perm(64, 8),
               _flat_perm(32), _flat_perm(16)]]

    w1big = jnp.transpose(fc1_w.reshape(64, 32, 64), (2, 1, 0)).reshape(64, 2048)
    diagm = jnp.repeat(jnp.eye(32, dtype=f32), 64, axis=1)

    G = _IMGS_PER_PROG
    full = lambda arr: pl.BlockSpec(arr.shape, lambda b: (0,) * arr.ndim)
    operands = [xf]
    in_specs = [pl.BlockSpec((G, 3, 256, 256), lambda b: (b, 0, 0, 0))]
    for wp, bc in zip(packed, bcols):
        operands += [wp, bc]
        in_specs += [full(wp), full(bc)]
    for d in d_mats:
        operands.append(d)
        in_specs.append(full(d))
    tail = [w1big.astype(bf16), diagm, fc1_b.reshape(1, 64).astype(f32),
            fc2_w.T.astype(bf16), fc2_b.reshape(1, out_dim).astype(f32)]
    for t in tail:
        operands.append(t)
        in_specs.append(full(t))

    out = pl.pallas_call(
        _fused_kernel,
        out_shape=jax.ShapeDtypeStruct((B, 1, out_dim), f32),
        grid_spec=pltpu.PrefetchScalarGridSpec(
            num_scalar_prefetch=0,
            grid=(B // G,),
            in_specs=in_specs,
            out_specs=pl.BlockSpec((G, 1, out_dim), lambda b: (b, 0, 0)),
        ),
        compiler_params=pltpu.CompilerParams(
            dimension_semantics=("parallel",),
            vmem_limit_bytes=100 * 1024 * 1024),
    )(*operands)
    return out.reshape(B, out_dim)


# speedup vs baseline: 4.2532x; 1.0383x over previous
import jax
import jax.numpy as jnp
from jax.experimental import pallas as pl
from jax.experimental.pallas import tpu as pltpu


def _space_to_depth_nchw(x):
    B, C, H, W = x.shape
    t = x.reshape(B, C, H // 2, 2, W // 2, 2)
    t = jnp.transpose(t, (0, 3, 5, 1, 2, 4))
    return t.reshape(B, 4 * C, (H // 2) * (W // 2))


def _pack_conv_weight(w):
    O, C = w.shape[0], w.shape[1]
    z = jnp.zeros((O, C), w.dtype)
    t = lambda kh, kw: w[:, :, kh, kw]
    wa = jnp.concatenate([t(1, 1), t(1, 2), t(2, 1), t(2, 2)], axis=1)
    wb = jnp.concatenate([z,       t(1, 0), z,       t(2, 0)], axis=1)
    wc = jnp.concatenate([z,       z,       t(0, 1), t(0, 2)], axis=1)
    wd = jnp.concatenate([z,       z,       z,       t(0, 0)], axis=1)
    return jnp.concatenate([wa, wb, wc, wd], axis=0)


def _perm_cols(src):
    n = src.shape[0]
    return jnp.eye(n, dtype=jnp.float32)[:, src]


def _packed_perm(S, k):
    n = k * S
    t = jnp.arange(n)
    blk = t // (n // 4)
    p, q = blk // 2, blk % 2
    r = t % (n // 4)
    j, w_ = r // (S // 2), r % (S // 2)
    return _perm_cols((2 * j + p) * S + 2 * w_ + q)


def _flat_perm(S):
    n = S * S
    t = jnp.arange(n)
    blk = t // (n // 4)
    p, q = blk // 2, blk % 2
    r = t % (n // 4)
    h_, w_ = r // (S // 2), r % (S // 2)
    return _perm_cols((2 * h_ + p) * S + 2 * w_ + q)


def _conv_block(z, w_packed, b_col, wo_out):
    f32 = jnp.float32
    M = z.shape[1]
    O = w_packed.shape[0] // 4
    t = jnp.dot(w_packed, z, preferred_element_type=f32)
    ya = t[0:O]
    yb = t[O:2 * O]
    yc = t[2 * O:3 * O]
    yd = t[3 * O:4 * O]
    col = jax.lax.broadcasted_iota(jnp.int32, (1, M), 1)
    wo_is0 = (col & (wo_out - 1)) == 0
    bd = yb + jnp.concatenate(
        [jnp.zeros((O, wo_out), f32), yd[:, :M - wo_out]], axis=1)
    sbd = jnp.where(
        wo_is0, 0.0,
        jnp.concatenate([jnp.zeros((O, 1), f32), bd[:, :M - 1]], axis=1))
    sc = jnp.concatenate(
        [jnp.zeros((O, wo_out), f32), yc[:, :M - wo_out]], axis=1)
    acc = ya + sbd + sc + b_col
    return jnp.maximum(acc.astype(jnp.bfloat16), 0)


_IMGS_PER_PROG = 4


def _fused_kernel(x_ref, w0, b0, w1, b1, w2, b2, w3, b3, w4, b4,
                  dx, dl0, dh0, d1, d2, d3, w1big, diagm, b1r, w2t, b2r,
                  o_ref):
    f32 = jnp.float32
    dot = lambda a, b: jnp.dot(a, b, preferred_element_type=f32)
    dotb = lambda a, b: jnp.dot(
        a, b, preferred_element_type=f32).astype(jnp.bfloat16)

    gs = range(_IMGS_PER_PROG)
    xa = [x_ref[g].astype(jnp.bfloat16).reshape(3, 65536).reshape(384, 512)
          for g in gs]
    ab = [[dotb(xa[g][:, p * 256:(p + 1) * 256], dx[...]) for p in range(2)]
          for g in gs]
    z = [jnp.concatenate(
        [ab[g][p][:, q * 128:(q + 1) * 128].reshape(3, 16384)
         for p in range(2) for q in range(2)], 0) for g in gs]
    y = [_conv_block(z[g], w0[...], b0[...], 128) for g in gs]

    yr = [y[g].reshape(512, 512) for g in gs]
    ab = [[dotb(yr[g][:, p * 128:(p + 1) * 128], dl0[...]) +
           dotb(yr[g][:, (p + 2) * 128:(p + 3) * 128], dh0[...])
           for p in range(2)] for g in gs]
    z = [jnp.concatenate(
        [ab[g][p][:, q * 128:(q + 1) * 128].reshape(16, 4096)
         for p in range(2) for q in range(2)], 0) for g in gs]
    y = [_conv_block(z[g], w1[...], b1[...], 64) for g in gs]

    a = [dotb(y[g].reshape(256, 512), d1[...]) for g in gs]
    z = [jnp.concatenate(
        [a[g][:, k * 128:(k + 1) * 128].reshape(32, 1024) for k in range(4)],
        0) for g in gs]
    y = [_conv_block(z[g], w2[...], b2[...], 32) for g in gs]

    a = [dotb(y[g], d2[...]) for g in gs]
    z = [jnp.concatenate(
        [a[g][:, k * 256:(k + 1) * 256] for k in range(4)], 0) for g in gs]
    y = [_conv_block(z[g], w3[...], b3[...], 16) for g in gs]

    a = [dotb(y[g], d3[...]) for g in gs]
    z = [jnp.concatenate(
        [a[g][:, k * 64:(k + 1) * 64] for k in range(4)], 0) for g in gs]
    y = [_conv_block(z[g], w4[...], b4[...], 8) for g in gs]

    for g in gs:
        v = dot(y[g], w1big[...]) * diagm[...]
        u = jnp.sum(v, axis=0, keepdims=True).reshape(16, 128)
        u = jnp.sum(u, axis=0, keepdims=True)
        h = u[:, :64] + u[:, 64:] + b1r[...]
        h = jnp.where(h > 0.0, h, 0.2 * h)
        o_ref[g] = dot(h.astype(jnp.bfloat16), w2t[...]) + b2r[...]


def kernel(x, conv0_w, conv0_b, conv1_w, conv1_b, conv2_w, conv2_b,
           conv3_w, conv3_b, conv4_w, conv4_b, fc1_w, fc1_b, fc2_w, fc2_b):
    f32 = jnp.float32
    bf16 = jnp.bfloat16
    B = x.shape[0]
    out_dim = fc2_w.shape[0]
    xf = x

    conv_ws = [conv0_w, conv1_w, conv2_w, conv3_w, conv4_w]
    conv_bs = [conv0_b, conv1_b, conv2_b, conv3_b, conv4_b]
    packed = [_pack_conv_weight(w).astype(bf16) for w in conv_ws]
    bcols = [b.reshape(-1, 1).astype(f32) for b in conv_bs]

    eye = lambda n: jnp.eye(n, dtype=f32)
    deint = lambda n, q: eye(n)[:, 2 * jnp.arange(n // 2) + q]
    zpad = jnp.zeros((128, 64), f32)
    dx = jnp.concatenate([deint(256, 0), deint(256, 1)], axis=1)
    dl0 = jnp.concatenate(
        [deint(128, 0), zpad, deint(128, 1), zpad], axis=1)
    dh0 = jnp.concatenate(
        [zpad, deint(128, 0), zpad, deint(128, 1)], axis=1)
    d_mats = [m.astype(bf16) for m in
              [dx, dl0, dh0, _packed_perm(64, 8),
               _flat_perm(32), _flat_perm(16)]]

    w1big = jnp.transpose(fc1_w.reshape(64, 32, 64), (2, 1, 0)).reshape(64, 2048)
    diagm = jnp.repeat(jnp.eye(32, dtype=f32), 64, axis=1)

    G = _IMGS_PER_PROG
    full = lambda arr: pl.BlockSpec(arr.shape, lambda b: (0,) * arr.ndim)
    operands = [xf]
    in_specs = [pl.BlockSpec((G, 3, 256, 256), lambda b: (b, 0, 0, 0))]
    for wp, bc in zip(packed, bcols):
        operands += [wp, bc]
        in_specs += [full(wp), full(bc)]
    for d in d_mats:
        operands.append(d)
        in_specs.append(full(d))
    tail = [w1big.astype(bf16), diagm, fc1_b.reshape(1, 64).astype(f32),
            fc2_w.T.astype(bf16), fc2_b.reshape(1, out_dim).astype(f32)]
    for t in tail:
        operands.append(t)
        in_specs.append(full(t))

    out = pl.pallas_call(
        _fused_kernel,
        out_shape=jax.ShapeDtypeStruct((B, 1, out_dim), f32),
        grid_spec=pltpu.PrefetchScalarGridSpec(
            num_scalar_prefetch=0,
            grid=(B // G,),
            in_specs=in_specs,
            out_specs=pl.BlockSpec((G, 1, out_dim), lambda b: (b, 0, 0)),
        ),
        compiler_params=pltpu.CompilerParams(
            dimension_semantics=("parallel",),
            vmem_limit_bytes=100 * 1024 * 1024),
    )(*operands)
    return out.reshape(B, out_dim)
